# baseline (speedup 1.0000x reference)
import functools

import jax
import jax.numpy as jnp
from jax import lax
from jax.experimental import pallas as pl
from jax.experimental.pallas import tpu as pltpu
from jax.experimental.pallas import tpu_sc as plsc

N = 10000
E = 160000
NC = 2
NS = 16
LANES = 128
NP = 10240
EP = 163840
EROWSP = EP // LANES
RPTP = NP // NS
DROWS = EROWSP // (NC * NS)
AROWS = EROWSP // NS
BLK = 512
GRID = NP // BLK


def _mesh():
    return plsc.VectorSubcoreMesh(
        core_axis_name="c", subcore_axis_name="s",
        num_cores=NC, num_subcores=NS)


_SC_PARAMS = pltpu.CompilerParams(use_tc_tiling_on_sc=False)


def _al(x):
    return pl.multiple_of(x, 8)


def _sc_degree(dst2d, ones, zeros):
    @functools.partial(
        pl.kernel,
        out_type=jax.ShapeDtypeStruct((2 * NP, 16), jnp.float32),
        mesh=_mesh(),
        compiler_params=_SC_PARAMS,
        scratch_types=[
            pltpu.VMEM((DROWS, LANES), jnp.int32),
            pltpu.VMEM((LANES, 16), jnp.float32),
            pltpu.VMEM_SHARED((NP, 16), jnp.float32),
        ],
    )
    def k(dst_hbm, ones_hbm, zeros_hbm, out_hbm, idx_v, ones_v, acc):
        cid = lax.axis_index("c")
        sid = lax.axis_index("s")
        w = sid * NC + cid
        pltpu.sync_copy(ones_hbm, ones_v)
        pltpu.sync_copy(dst_hbm.at[pl.ds(_al(w * DROWS), DROWS)], idx_v)
        pltpu.sync_copy(zeros_hbm, acc.at[pl.ds(_al(sid * RPTP), RPTP)])
        plsc.subcore_barrier()

        def body(j, c):
            pltpu.sync_copy(ones_v, acc.at[idx_v.at[j]], add=True)
            return c

        lax.fori_loop(0, DROWS, body, 0)
        plsc.subcore_barrier()
        pltpu.sync_copy(acc.at[pl.ds(_al(sid * RPTP), RPTP)],
                        out_hbm.at[pl.ds(_al(cid * NP + sid * RPTP), RPTP)])

    return k(dst2d, ones, zeros)


def _sc_agg(u1, u2, e2d, wc):
    @functools.partial(
        pl.kernel,
        out_type=(jax.ShapeDtypeStruct((NP, wc), jnp.float32),
                  jax.ShapeDtypeStruct((NP, wc), jnp.float32)),
        mesh=_mesh(),
        compiler_params=_SC_PARAMS,
        scratch_types=[
            pltpu.VMEM((8, 2, LANES), jnp.int32),
            pltpu.VMEM((LANES, wc), jnp.float32),
            pltpu.VMEM((LANES, wc), jnp.float32),
            pltpu.VMEM_SHARED((NP, wc), jnp.float32),
            pltpu.SemaphoreType.DMA,
            pltpu.SemaphoreType.DMA,
            pltpu.SemaphoreType.DMA,
            pltpu.SemaphoreType.DMA,
        ],
    )
    def k(u1_hbm, u2_hbm, edges, out1, out2, exv, rv0, rv1, acc,
          sg0, sg1, ss0, ss1):
        cid = lax.axis_index("c")
        sid = lax.axis_index("s")
        sl = pl.ds(_al(sid * RPTP), RPTP)

        @pl.when(cid == 0)
        def _():
            pltpu.sync_copy(u1_hbm.at[sl], acc.at[sl])

        @pl.when(cid == 1)
        def _():
            pltpu.sync_copy(u2_hbm.at[sl], acc.at[sl])

        plsc.subcore_barrier()
        r0 = sid * AROWS

        def gstart(jj, rv, sg):
            @pl.when(cid == 0)
            def _():
                pltpu.async_copy(u1_hbm.at[exv.at[jj, 0]], rv, sg)

            @pl.when(cid == 1)
            def _():
                pltpu.async_copy(u2_hbm.at[exv.at[jj, 0]], rv, sg)

        def gwait(jj, rv, sg):
            pltpu.make_async_copy(u1_hbm.at[exv.at[jj, 0]], rv, sg).wait()

        def sstart(jj, rv, ss):
            pltpu.async_copy(rv, acc.at[exv.at[jj, 1]], ss, add=True)

        def swait(jj, rv, ss):
            pltpu.make_async_copy(rv, acc.at[exv.at[jj, 1]], ss).wait()

        def group(g, c):
            pltpu.sync_copy(edges.at[pl.ds(_al(r0 + g * 8), 8)], exv)
            gstart(0, rv0, sg0)
            gstart(1, rv1, sg1)

            def pair(k2, c2):
                j0 = 2 * k2
                j1 = j0 + 1
                gwait(j0, rv0, sg0)
                sstart(j0, rv0, ss0)
                gwait(j1, rv1, sg1)
                sstart(j1, rv1, ss1)

                @pl.when(j0 + 2 < 8)
                def _():
                    swait(j0, rv0, ss0)
                    gstart(j0 + 2, rv0, sg0)
                    swait(j1, rv1, ss1)
                    gstart(j1 + 2, rv1, sg1)

                return c2

            lax.fori_loop(0, 4, pair, 0)
            swait(6, rv0, ss0)
            swait(7, rv1, ss1)
            return c

        lax.fori_loop(0, AROWS // 8, group, 0)
        plsc.subcore_barrier()

        @pl.when(cid == 0)
        def _():
            pltpu.sync_copy(acc.at[sl], out1.at[sl])

        @pl.when(cid == 1)
        def _():
            pltpu.sync_copy(acc.at[sl], out2.at[sl])

    return k(u1, u2, e2d)


def _sc_agg2(u1, u2, s2d, d2d, wc):
    W2 = 4 * LANES
    SLABS = AROWS // 4

    @functools.partial(
        pl.kernel,
        out_type=(jax.ShapeDtypeStruct((NP, wc), jnp.float32),
                  jax.ShapeDtypeStruct((NP, wc), jnp.float32)),
        mesh=_mesh(),
        compiler_params=_SC_PARAMS,
        scratch_types=[
            pltpu.VMEM((SLABS, W2), jnp.int32),
            pltpu.VMEM((SLABS, W2), jnp.int32),
            pltpu.VMEM((W2, wc), jnp.float32),
            pltpu.VMEM((W2, wc), jnp.float32),
            pltpu.VMEM_SHARED((NP, wc), jnp.float32),
            pltpu.SemaphoreType.DMA,
            pltpu.SemaphoreType.DMA,
            pltpu.SemaphoreType.DMA,
            pltpu.SemaphoreType.DMA,
        ],
    )
    def k(u1_hbm, u2_hbm, s2d_h, d2d_h, out1, out2, sidx, didx, rv0, rv1,
          acc, sg0, sg1, ss0, ss1):
        cid = lax.axis_index("c")
        sid = lax.axis_index("s")
        sl = pl.ds(_al(sid * RPTP), RPTP)

        @pl.when(cid == 0)
        def _():
            pltpu.sync_copy(u1_hbm.at[sl], acc.at[sl])

        @pl.when(cid == 1)
        def _():
            pltpu.sync_copy(u2_hbm.at[sl], acc.at[sl])

        r0 = _al(sid * SLABS)
        pltpu.sync_copy(s2d_h.at[pl.ds(r0, SLABS)], sidx)
        pltpu.sync_copy(d2d_h.at[pl.ds(r0, SLABS)], didx)
        plsc.subcore_barrier()

        def gstart(j, rv, sg):
            @pl.when(cid == 0)
            def _():
                pltpu.async_copy(u1_hbm.at[sidx.at[j]], rv, sg)

            @pl.when(cid == 1)
            def _():
                pltpu.async_copy(u2_hbm.at[sidx.at[j]], rv, sg)

        def gwait(j, rv, sg):
            pltpu.make_async_copy(u1_hbm.at[sidx.at[j]], rv, sg).wait()

        def sstart(j, rv, ss):
            pltpu.async_copy(rv, acc.at[didx.at[j]], ss, add=True)

        def swait(j, rv, ss):
            pltpu.make_async_copy(rv, acc.at[didx.at[j]], ss).wait()

        gstart(0, rv0, sg0)
        gstart(1, rv1, sg1)

        def pair(k2, c):
            j0 = 2 * k2
            j1 = j0 + 1
            gwait(j0, rv0, sg0)
            sstart(j0, rv0, ss0)
            gwait(j1, rv1, sg1)
            sstart(j1, rv1, ss1)

            @pl.when(j0 + 2 < SLABS)
            def _():
                swait(j0, rv0, ss0)
                gstart(j0 + 2, rv0, sg0)
                swait(j1, rv1, ss1)
                gstart(j1 + 2, rv1, sg1)

            return c

        lax.fori_loop(0, SLABS // 2, pair, 0)
        swait(SLABS - 2, rv0, ss0)
        swait(SLABS - 1, rv1, ss1)
        plsc.subcore_barrier()

        @pl.when(cid == 0)
        def _():
            pltpu.sync_copy(acc.at[sl], out1.at[sl])

        @pl.when(cid == 1)
        def _():
            pltpu.sync_copy(acc.at[sl], out2.at[sl])

    return k(u1, u2, s2d, d2d)


def _sc_gather_edges(z, src2d, dst2d):
    @functools.partial(
        pl.kernel,
        out_type=(jax.ShapeDtypeStruct((EP, 32), jnp.float32),
                  jax.ShapeDtypeStruct((EP, 32), jnp.float32)),
        mesh=_mesh(),
        compiler_params=_SC_PARAMS,
        scratch_types=[
            pltpu.VMEM((DROWS // 4, 4 * LANES), jnp.int32),
            pltpu.VMEM((DROWS // 4, 4 * LANES), jnp.int32),
            pltpu.VMEM((4 * LANES, 32), jnp.float32),
            pltpu.VMEM((4 * LANES, 32), jnp.float32),
            pltpu.VMEM((4 * LANES, 32), jnp.float32),
            pltpu.VMEM((4 * LANES, 32), jnp.float32),
            pltpu.SemaphoreType.DMA,
            pltpu.SemaphoreType.DMA,
            pltpu.SemaphoreType.DMA,
            pltpu.SemaphoreType.DMA,
        ],
    )
    def k(z_hbm, s2d, d2d, zs_out, zd_out, sidx, didx,
          zs0, zd0, zs1, zd1, sa0, sb0, sa1, sb1):
        R = 4
        SLABS = DROWS // R
        cid = lax.axis_index("c")
        sid = lax.axis_index("s")
        w = sid * NC + cid
        r0 = _al(w * SLABS)
        pltpu.sync_copy(s2d.at[pl.ds(r0, SLABS)], sidx)
        pltpu.sync_copy(d2d.at[pl.ds(r0, SLABS)], didx)

        def gstart(j, zsb, zdb, sa, sb):
            pltpu.async_copy(z_hbm.at[sidx.at[j]], zsb, sa)
            pltpu.async_copy(z_hbm.at[didx.at[j]], zdb, sb)

        def drain(j, zsb, zdb, sa, sb):
            pltpu.make_async_copy(z_hbm.at[sidx.at[j]], zsb, sa).wait()
            pltpu.make_async_copy(z_hbm.at[didx.at[j]], zdb, sb).wait()
            esl = pl.ds(_al((r0 + j) * R * LANES), R * LANES)
            pltpu.sync_copy(zsb, zs_out.at[esl])
            pltpu.sync_copy(zdb, zd_out.at[esl])

        gstart(0, zs0, zd0, sa0, sb0)

        def pair(k2, c):
            j0 = 2 * k2
            j1 = j0 + 1
            gstart(j1, zs1, zd1, sa1, sb1)
            drain(j0, zs0, zd0, sa0, sb0)

            @pl.when(j0 + 2 < SLABS)
            def _():
                gstart(j0 + 2, zs0, zd0, sa0, sb0)

            drain(j1, zs1, zd1, sa1, sb1)
            return c

        lax.fori_loop(0, SLABS // 2, pair, 0)

    return k(z, src2d, dst2d)


def _rows(i):
    return (i, 0)


def _const(i):
    return (0, 0)


def _tc_front(parts3, x1, x2, w11, w21):
    def body(p0_r, p1_r, x1_r, w1_r, x2_r, w2_r, u1_r, u2_r, dinv_r):
        deg = p0_r[0, :, 0:1] + p1_r[0, :, 0:1] + 1.0
        dinv = jnp.where(deg > 0, 1.0 / jnp.sqrt(deg), 0.0)
        dinv_r[...] = dinv
        u1_r[...] = jnp.dot(x1_r[...], w1_r[...],
                            preferred_element_type=jnp.float32) * dinv
        u2_r[...] = jnp.dot(x2_r[...], w2_r[...],
                            preferred_element_type=jnp.float32) * dinv

    d = x1.shape[1]
    h = w11.shape[1]
    return pl.pallas_call(
        body,
        grid=(GRID,),
        in_specs=[pl.BlockSpec((1, BLK, 16), lambda i: (0, i, 0)),
                  pl.BlockSpec((1, BLK, 16), lambda i: (1, i, 0)),
                  pl.BlockSpec((BLK, d), _rows), pl.BlockSpec((d, h), _const),
                  pl.BlockSpec((BLK, d), _rows), pl.BlockSpec((d, h), _const)],
        out_specs=[pl.BlockSpec((BLK, h), _rows), pl.BlockSpec((BLK, h), _rows),
                   pl.BlockSpec((BLK, 1), _rows)],
        out_shape=[jax.ShapeDtypeStruct((NP, h), jnp.float32),
                   jax.ShapeDtypeStruct((NP, h), jnp.float32),
                   jax.ShapeDtypeStruct((NP, 1), jnp.float32)],
    )(parts3, parts3, x1, w11, x2, w21)


def _tc_mid(a1, a2, dinv, b11, b21, w12, w22):
    def body(a1_r, a2_r, di_r, b1_r, b2_r, w1_r, w2_r, v1_r, v2_r):
        di = di_r[...]
        g1 = jnp.maximum(a1_r[...] * di + b1_r[...], 0.0)
        g2 = jnp.maximum(a2_r[...] * di + b2_r[...], 0.0)
        v1_r[...] = jnp.dot(g1, w1_r[...],
                            preferred_element_type=jnp.float32) * di
        v2_r[...] = jnp.dot(g2, w2_r[...],
                            preferred_element_type=jnp.float32) * di

    h1 = a1.shape[1]
    h2 = w12.shape[1]
    return pl.pallas_call(
        body,
        grid=(GRID,),
        in_specs=[pl.BlockSpec((BLK, h1), _rows), pl.BlockSpec((BLK, h1), _rows),
                  pl.BlockSpec((BLK, 1), _rows),
                  pl.BlockSpec((1, h1), _const), pl.BlockSpec((1, h1), _const),
                  pl.BlockSpec((h1, h2), _const), pl.BlockSpec((h1, h2), _const)],
        out_specs=[pl.BlockSpec((BLK, h2), _rows), pl.BlockSpec((BLK, h2), _rows)],
        out_shape=[jax.ShapeDtypeStruct((NP, h2), jnp.float32)] * 2,
    )(a1, a2, dinv, b11, b21, w12, w22)


def _tc_latent(c1, c2, dinv, eps, p):
    def body(c1_r, c2_r, di_r, eps_r, b12_r, b22_r, fw_r, fb_r,
             muw_r, mub_r, lvw_r, lvb_r, z_r, mu_r, lv_r):
        di = di_r[...]
        h1 = jnp.maximum(c1_r[...] * di + b12_r[...], 0.0)
        h2 = jnp.maximum(c2_r[...] * di + b22_r[...], 0.0)
        hcat = jnp.concatenate([h1, h2], axis=1)
        h = jnp.maximum(jnp.dot(hcat, fw_r[...],
                                preferred_element_type=jnp.float32)
                        + fb_r[...], 0.0)
        mu = jnp.dot(h, muw_r[...], preferred_element_type=jnp.float32) \
            + mub_r[...]
        lv = jnp.dot(h, lvw_r[...], preferred_element_type=jnp.float32) \
            + lvb_r[...]
        z_r[...] = mu + eps_r[...] * jnp.exp(0.5 * lv)
        mu_r[...] = mu
        lv_r[...] = lv

    h2 = c1.shape[1]
    fin = 2 * h2
    fus = p['fuse_w'].shape[1]
    zd = p['mu_w'].shape[1]

    def cs(shape):
        return pl.BlockSpec(shape, _const)

    return pl.pallas_call(
        body,
        grid=(GRID,),
        in_specs=[pl.BlockSpec((BLK, h2), _rows), pl.BlockSpec((BLK, h2), _rows),
                  pl.BlockSpec((BLK, 1), _rows), pl.BlockSpec((BLK, zd), _rows),
                  cs((1, h2)), cs((1, h2)),
                  cs((fin, fus)), cs((1, fus)),
                  cs((fus, zd)), cs((1, zd)),
                  cs((fus, zd)), cs((1, zd))],
        out_specs=[pl.BlockSpec((BLK, zd), _rows), pl.BlockSpec((BLK, zd), _rows),
                   pl.BlockSpec((BLK, zd), _rows)],
        out_shape=[jax.ShapeDtypeStruct((NP, zd), jnp.float32)] * 3,
    )(c1, c2, dinv, eps,
      p['gcn1_b2'].reshape(1, h2), p['gcn2_b2'].reshape(1, h2),
      p['fuse_w'], p['fuse_b'].reshape(1, fus),
      p['mu_w'], p['mu_b'].reshape(1, zd),
      p['logvar_w'], p['logvar_b'].reshape(1, zd))


def _tc_decode(z, p):
    def body(z_r, r1w1_r, r1b1_r, r1w2_r, r1b2_r,
             r2w1_r, r2b1_r, r2w2_r, r2b2_r,
             rsw1_r, rsb1_r, rsw2_r, rsb2_r, x1_r, x2_r, xs_r):
        z_b = z_r[...]
        t1 = jnp.maximum(jnp.dot(z_b, r1w1_r[...],
                                 preferred_element_type=jnp.float32)
                         + r1b1_r[...], 0.0)
        x1_r[...] = jnp.dot(t1, r1w2_r[...],
                            preferred_element_type=jnp.float32) + r1b2_r[...]
        t2 = jnp.maximum(jnp.dot(z_b, r2w1_r[...],
                                 preferred_element_type=jnp.float32)
                         + r2b1_r[...], 0.0)
        x2_r[...] = jnp.dot(t2, r2w2_r[...],
                            preferred_element_type=jnp.float32) + r2b2_r[...]
        ts = jnp.maximum(jnp.dot(z_b, rsw1_r[...],
                                 preferred_element_type=jnp.float32)
                         + rsb1_r[...], 0.0)
        xs_r[...] = jnp.dot(ts, rsw2_r[...],
                            preferred_element_type=jnp.float32) + rsb2_r[...]

    fus = p['fuse_w'].shape[1]
    zd = p['mu_w'].shape[1]
    r1 = p['rec1_w2'].shape[1]
    r2 = p['rec2_w2'].shape[1]
    rs = p['recs_w2'].shape[1]

    def cs(shape):
        return pl.BlockSpec(shape, _const)

    return pl.pallas_call(
        body,
        grid=(GRID,),
        in_specs=[pl.BlockSpec((BLK, zd), _rows),
                  cs((zd, fus)), cs((1, fus)), cs((fus, r1)), cs((1, r1)),
                  cs((zd, fus)), cs((1, fus)), cs((fus, r2)), cs((1, r2)),
                  cs((zd, fus)), cs((1, fus)), cs((fus, rs)), cs((1, rs))],
        out_specs=[pl.BlockSpec((BLK, r1), _rows), pl.BlockSpec((BLK, r2), _rows),
                   pl.BlockSpec((BLK, rs), _rows)],
        out_shape=[jax.ShapeDtypeStruct((NP, r1), jnp.float32),
                   jax.ShapeDtypeStruct((NP, r2), jnp.float32),
                   jax.ShapeDtypeStruct((NP, rs), jnp.float32)],
    )(z, p['rec1_w1'], p['rec1_b1'].reshape(1, fus),
      p['rec1_w2'], p['rec1_b2'].reshape(1, r1),
      p['rec2_w1'], p['rec2_b1'].reshape(1, fus),
      p['rec2_w2'], p['rec2_b2'].reshape(1, r2),
      p['recs_w1'], p['recs_b1'].reshape(1, fus),
      p['recs_w2'], p['recs_b2'].reshape(1, rs))


def _tc_edge_dot(zs, zd):
    eb = 2048
    egrid = EP // eb

    def body(zs_r, zd_r, o_r):
        s = jnp.sum(zs_r[...] * zd_r[...], axis=1, keepdims=True)
        o_r[...] = jax.nn.sigmoid(s)

    k = zs.shape[1]
    return pl.pallas_call(
        body,
        grid=(egrid,),
        in_specs=[pl.BlockSpec((eb, k), _rows), pl.BlockSpec((eb, k), _rows)],
        out_specs=pl.BlockSpec((eb, 1), _rows),
        out_shape=jax.ShapeDtypeStruct((EP, 1), jnp.float32),
    )(zs, zd)


def kernel(x_omics1, x_omics2, edge_index, params):
    p = params
    src = edge_index[0].astype(jnp.int32)
    dst = edge_index[1].astype(jnp.int32)
    pad = EP - E
    src2d = jnp.concatenate([src, jnp.zeros((pad,), jnp.int32)])
    src2d = src2d.reshape(EROWSP, LANES)
    dst2d = jnp.concatenate([dst, jnp.full((pad,), N, jnp.int32)])
    dst2d = dst2d.reshape(EROWSP, LANES)
    ones = jnp.ones((LANES, 16), jnp.float32)
    zeros = jnp.zeros((RPTP, 16), jnp.float32)
    e2d = jnp.stack([src2d, dst2d], axis=1)
    s512 = src2d.reshape(EROWSP // 4, 4 * LANES)
    d512 = dst2d.reshape(EROWSP // 4, 4 * LANES)
    eps = jax.random.normal(jax.random.key(42), (N, 32), jnp.float32)

    parts = _sc_degree(dst2d, ones, zeros)
    u1, u2, dinv = _tc_front(parts.reshape(2, NP, 16), x_omics1, x_omics2,
                             p['gcn1_w1'], p['gcn2_w1'])
    a1, a2 = _sc_agg(u1, u2, e2d, u1.shape[1])
    v1, v2 = _tc_mid(a1, a2, dinv, p['gcn1_b1'].reshape(1, -1),
                     p['gcn2_b1'].reshape(1, -1), p['gcn1_w2'], p['gcn2_w2'])
    c1, c2 = _sc_agg2(v1, v2, s512, d512, v1.shape[1])
    z, mu, logvar = _tc_latent(c1, c2, dinv, eps, p)
    zs, zd = _sc_gather_edges(z, s512, d512)
    xh1, xh2, xhs = _tc_decode(z, p)
    adj = _tc_edge_dot(zs, zd).reshape(EP)[:E]
    return (z[:N], mu[:N], logvar[:N], xh1[:N], xh2[:N], xhs[:N], adj)


# speedup vs baseline: 1.0240x; 1.0240x over previous
import functools

import jax
import jax.numpy as jnp
from jax import lax
from jax.experimental import pallas as pl
from jax.experimental.pallas import tpu as pltpu
from jax.experimental.pallas import tpu_sc as plsc

N = 10000
E = 160000
NC = 2
NS = 16
LANES = 128
NP = 10240
EP = 163840
EROWSP = EP // LANES
RPTP = NP // NS
DROWS = EROWSP // (NC * NS)
AROWS = EROWSP // NS
BLK = 512
GRID = NP // BLK


def _mesh():
    return plsc.VectorSubcoreMesh(
        core_axis_name="c", subcore_axis_name="s",
        num_cores=NC, num_subcores=NS)


_SC_PARAMS = pltpu.CompilerParams(use_tc_tiling_on_sc=False,
                                  skip_device_barrier=True)


def _al(x):
    return pl.multiple_of(x, 8)


def _sc_degree(dst2d, ones, zeros):
    @functools.partial(
        pl.kernel,
        out_type=jax.ShapeDtypeStruct((2 * NP, 16), jnp.float32),
        mesh=_mesh(),
        compiler_params=_SC_PARAMS,
        scratch_types=[
            pltpu.VMEM((DROWS, LANES), jnp.int32),
            pltpu.VMEM((LANES, 16), jnp.float32),
            pltpu.VMEM_SHARED((NP, 16), jnp.float32),
        ],
    )
    def k(dst_hbm, ones_hbm, zeros_hbm, out_hbm, idx_v, ones_v, acc):
        cid = lax.axis_index("c")
        sid = lax.axis_index("s")
        w = sid * NC + cid
        pltpu.sync_copy(ones_hbm, ones_v)
        pltpu.sync_copy(dst_hbm.at[pl.ds(_al(w * DROWS), DROWS)], idx_v)
        pltpu.sync_copy(zeros_hbm, acc.at[pl.ds(_al(sid * RPTP), RPTP)])
        plsc.subcore_barrier()

        def body(j, c):
            pltpu.sync_copy(ones_v, acc.at[idx_v.at[j]], add=True)
            return c

        lax.fori_loop(0, DROWS, body, 0)
        plsc.subcore_barrier()
        pltpu.sync_copy(acc.at[pl.ds(_al(sid * RPTP), RPTP)],
                        out_hbm.at[pl.ds(_al(cid * NP + sid * RPTP), RPTP)])

    return k(dst2d, ones, zeros)


def _sc_agg(u1, u2, e2d, wc):
    @functools.partial(
        pl.kernel,
        out_type=(jax.ShapeDtypeStruct((NP, wc), jnp.float32),
                  jax.ShapeDtypeStruct((NP, wc), jnp.float32)),
        mesh=_mesh(),
        compiler_params=_SC_PARAMS,
        scratch_types=[
            pltpu.VMEM((8, 2, LANES), jnp.int32),
            pltpu.VMEM((LANES, wc), jnp.float32),
            pltpu.VMEM((LANES, wc), jnp.float32),
            pltpu.VMEM_SHARED((NP, wc), jnp.float32),
            pltpu.SemaphoreType.DMA,
            pltpu.SemaphoreType.DMA,
        ],
    )
    def k(u1_hbm, u2_hbm, edges, out1, out2, exv, rv0, rv1, acc, sg0, sg1):
        cid = lax.axis_index("c")
        sid = lax.axis_index("s")
        sl = pl.ds(_al(sid * RPTP), RPTP)

        @pl.when(cid == 0)
        def _():
            pltpu.sync_copy(u1_hbm.at[sl], acc.at[sl])

        @pl.when(cid == 1)
        def _():
            pltpu.sync_copy(u2_hbm.at[sl], acc.at[sl])

        plsc.subcore_barrier()
        r0 = sid * AROWS

        def gstart(jj, rv, sg):
            @pl.when(cid == 0)
            def _():
                pltpu.async_copy(u1_hbm.at[exv.at[jj, 0]], rv, sg)

            @pl.when(cid == 1)
            def _():
                pltpu.async_copy(u2_hbm.at[exv.at[jj, 0]], rv, sg)

        def gwait(jj, rv, sg):
            pltpu.make_async_copy(u1_hbm.at[exv.at[jj, 0]], rv, sg).wait()

        def group(g, c):
            pltpu.sync_copy(edges.at[pl.ds(_al(r0 + g * 8), 8)], exv)
            gstart(0, rv0, sg0)

            def pair(k2, c2):
                j0 = 2 * k2
                j1 = j0 + 1
                gstart(j1, rv1, sg1)
                gwait(j0, rv0, sg0)
                pltpu.sync_copy(rv0, acc.at[exv.at[j0, 1]], add=True)

                @pl.when(j0 + 2 < 8)
                def _():
                    gstart(j0 + 2, rv0, sg0)

                gwait(j1, rv1, sg1)
                pltpu.sync_copy(rv1, acc.at[exv.at[j1, 1]], add=True)
                return c2

            lax.fori_loop(0, 4, pair, 0)
            return c

        lax.fori_loop(0, AROWS // 8, group, 0)
        plsc.subcore_barrier()

        @pl.when(cid == 0)
        def _():
            pltpu.sync_copy(acc.at[sl], out1.at[sl])

        @pl.when(cid == 1)
        def _():
            pltpu.sync_copy(acc.at[sl], out2.at[sl])

    return k(u1, u2, e2d)


def _sc_agg2(u1, u2, s2d, d2d, wc):
    W2 = 4 * LANES
    SLABS = AROWS // 4

    @functools.partial(
        pl.kernel,
        out_type=(jax.ShapeDtypeStruct((NP, wc), jnp.float32),
                  jax.ShapeDtypeStruct((NP, wc), jnp.float32)),
        mesh=_mesh(),
        compiler_params=_SC_PARAMS,
        scratch_types=[
            pltpu.VMEM((SLABS, W2), jnp.int32),
            pltpu.VMEM((SLABS, W2), jnp.int32),
            pltpu.VMEM((W2, wc), jnp.float32),
            pltpu.VMEM((W2, wc), jnp.float32),
            pltpu.VMEM_SHARED((NP, wc), jnp.float32),
            pltpu.SemaphoreType.DMA,
            pltpu.SemaphoreType.DMA,
            pltpu.SemaphoreType.DMA,
            pltpu.SemaphoreType.DMA,
        ],
    )
    def k(u1_hbm, u2_hbm, s2d_h, d2d_h, out1, out2, sidx, didx, rv0, rv1,
          acc, sg0, sg1, ss0, ss1):
        cid = lax.axis_index("c")
        sid = lax.axis_index("s")
        sl = pl.ds(_al(sid * RPTP), RPTP)

        @pl.when(cid == 0)
        def _():
            pltpu.sync_copy(u1_hbm.at[sl], acc.at[sl])

        @pl.when(cid == 1)
        def _():
            pltpu.sync_copy(u2_hbm.at[sl], acc.at[sl])

        r0 = _al(sid * SLABS)
        pltpu.sync_copy(s2d_h.at[pl.ds(r0, SLABS)], sidx)
        pltpu.sync_copy(d2d_h.at[pl.ds(r0, SLABS)], didx)
        plsc.subcore_barrier()

        def gstart(j, rv, sg):
            @pl.when(cid == 0)
            def _():
                pltpu.async_copy(u1_hbm.at[sidx.at[j]], rv, sg)

            @pl.when(cid == 1)
            def _():
                pltpu.async_copy(u2_hbm.at[sidx.at[j]], rv, sg)

        def gwait(j, rv, sg):
            pltpu.make_async_copy(u1_hbm.at[sidx.at[j]], rv, sg).wait()

        def sstart(j, rv, ss):
            pltpu.async_copy(rv, acc.at[didx.at[j]], ss, add=True)

        def swait(j, rv, ss):
            pltpu.make_async_copy(rv, acc.at[didx.at[j]], ss).wait()

        gstart(0, rv0, sg0)
        gstart(1, rv1, sg1)

        def pair(k2, c):
            j0 = 2 * k2
            j1 = j0 + 1
            gwait(j0, rv0, sg0)
            sstart(j0, rv0, ss0)
            gwait(j1, rv1, sg1)
            sstart(j1, rv1, ss1)

            @pl.when(j0 + 2 < SLABS)
            def _():
                swait(j0, rv0, ss0)
                gstart(j0 + 2, rv0, sg0)
                swait(j1, rv1, ss1)
                gstart(j1 + 2, rv1, sg1)

            return c

        lax.fori_loop(0, SLABS // 2, pair, 0)
        swait(SLABS - 2, rv0, ss0)
        swait(SLABS - 1, rv1, ss1)
        plsc.subcore_barrier()

        @pl.when(cid == 0)
        def _():
            pltpu.sync_copy(acc.at[sl], out1.at[sl])

        @pl.when(cid == 1)
        def _():
            pltpu.sync_copy(acc.at[sl], out2.at[sl])

    return k(u1, u2, s2d, d2d)


def _sc_gather_edges(z, src2d, dst2d):
    @functools.partial(
        pl.kernel,
        out_type=(jax.ShapeDtypeStruct((EP, 32), jnp.float32),
                  jax.ShapeDtypeStruct((EP, 32), jnp.float32)),
        mesh=_mesh(),
        compiler_params=_SC_PARAMS,
        scratch_types=[
            pltpu.VMEM((DROWS // 4, 4 * LANES), jnp.int32),
            pltpu.VMEM((DROWS // 4, 4 * LANES), jnp.int32),
            pltpu.VMEM((4 * LANES, 32), jnp.float32),
            pltpu.VMEM((4 * LANES, 32), jnp.float32),
            pltpu.VMEM((4 * LANES, 32), jnp.float32),
            pltpu.VMEM((4 * LANES, 32), jnp.float32),
            pltpu.SemaphoreType.DMA,
            pltpu.SemaphoreType.DMA,
            pltpu.SemaphoreType.DMA,
            pltpu.SemaphoreType.DMA,
        ],
    )
    def k(z_hbm, s2d, d2d, zs_out, zd_out, sidx, didx,
          zs0, zd0, zs1, zd1, sa0, sb0, sa1, sb1):
        R = 4
        SLABS = DROWS // R
        cid = lax.axis_index("c")
        sid = lax.axis_index("s")
        w = sid * NC + cid
        r0 = _al(w * SLABS)
        pltpu.sync_copy(s2d.at[pl.ds(r0, SLABS)], sidx)
        pltpu.sync_copy(d2d.at[pl.ds(r0, SLABS)], didx)

        def gstart(j, zsb, zdb, sa, sb):
            pltpu.async_copy(z_hbm.at[sidx.at[j]], zsb, sa)
            pltpu.async_copy(z_hbm.at[didx.at[j]], zdb, sb)

        def drain(j, zsb, zdb, sa, sb):
            pltpu.make_async_copy(z_hbm.at[sidx.at[j]], zsb, sa).wait()
            pltpu.make_async_copy(z_hbm.at[didx.at[j]], zdb, sb).wait()
            esl = pl.ds(_al((r0 + j) * R * LANES), R * LANES)
            pltpu.sync_copy(zsb, zs_out.at[esl])
            pltpu.sync_copy(zdb, zd_out.at[esl])

        gstart(0, zs0, zd0, sa0, sb0)

        def pair(k2, c):
            j0 = 2 * k2
            j1 = j0 + 1
            gstart(j1, zs1, zd1, sa1, sb1)
            drain(j0, zs0, zd0, sa0, sb0)

            @pl.when(j0 + 2 < SLABS)
            def _():
                gstart(j0 + 2, zs0, zd0, sa0, sb0)

            drain(j1, zs1, zd1, sa1, sb1)
            return c

        lax.fori_loop(0, SLABS // 2, pair, 0)

    return k(z, src2d, dst2d)


def _rows(i):
    return (i, 0)


def _const(i):
    return (0, 0)


def _tc_front(parts3, x1, x2, w11, w21):
    def body(p0_r, p1_r, x1_r, w1_r, x2_r, w2_r, u1_r, u2_r, dinv_r):
        deg = p0_r[0, :, 0:1] + p1_r[0, :, 0:1] + 1.0
        dinv = jnp.where(deg > 0, 1.0 / jnp.sqrt(deg), 0.0)
        dinv_r[...] = dinv
        u1_r[...] = jnp.dot(x1_r[...], w1_r[...],
                            preferred_element_type=jnp.float32) * dinv
        u2_r[...] = jnp.dot(x2_r[...], w2_r[...],
                            preferred_element_type=jnp.float32) * dinv

    d = x1.shape[1]
    h = w11.shape[1]
    return pl.pallas_call(
        body,
        grid=(GRID,),
        in_specs=[pl.BlockSpec((1, BLK, 16), lambda i: (0, i, 0)),
                  pl.BlockSpec((1, BLK, 16), lambda i: (1, i, 0)),
                  pl.BlockSpec((BLK, d), _rows), pl.BlockSpec((d, h), _const),
                  pl.BlockSpec((BLK, d), _rows), pl.BlockSpec((d, h), _const)],
        out_specs=[pl.BlockSpec((BLK, h), _rows), pl.BlockSpec((BLK, h), _rows),
                   pl.BlockSpec((BLK, 1), _rows)],
        out_shape=[jax.ShapeDtypeStruct((NP, h), jnp.float32),
                   jax.ShapeDtypeStruct((NP, h), jnp.float32),
                   jax.ShapeDtypeStruct((NP, 1), jnp.float32)],
    )(parts3, parts3, x1, w11, x2, w21)


def _tc_mid(a1, a2, dinv, b11, b21, w12, w22):
    def body(a1_r, a2_r, di_r, b1_r, b2_r, w1_r, w2_r, v1_r, v2_r):
        di = di_r[...]
        g1 = jnp.maximum(a1_r[...] * di + b1_r[...], 0.0)
        g2 = jnp.maximum(a2_r[...] * di + b2_r[...], 0.0)
        v1_r[...] = jnp.dot(g1, w1_r[...],
                            preferred_element_type=jnp.float32) * di
        v2_r[...] = jnp.dot(g2, w2_r[...],
                            preferred_element_type=jnp.float32) * di

    h1 = a1.shape[1]
    h2 = w12.shape[1]
    return pl.pallas_call(
        body,
        grid=(GRID,),
        in_specs=[pl.BlockSpec((BLK, h1), _rows), pl.BlockSpec((BLK, h1), _rows),
                  pl.BlockSpec((BLK, 1), _rows),
                  pl.BlockSpec((1, h1), _const), pl.BlockSpec((1, h1), _const),
                  pl.BlockSpec((h1, h2), _const), pl.BlockSpec((h1, h2), _const)],
        out_specs=[pl.BlockSpec((BLK, h2), _rows), pl.BlockSpec((BLK, h2), _rows)],
        out_shape=[jax.ShapeDtypeStruct((NP, h2), jnp.float32)] * 2,
    )(a1, a2, dinv, b11, b21, w12, w22)


def _tc_latent(c1, c2, dinv, eps, p):
    def body(c1_r, c2_r, di_r, eps_r, b12_r, b22_r, fw_r, fb_r,
             muw_r, mub_r, lvw_r, lvb_r, z_r, mu_r, lv_r):
        di = di_r[...]
        h1 = jnp.maximum(c1_r[...] * di + b12_r[...], 0.0)
        h2 = jnp.maximum(c2_r[...] * di + b22_r[...], 0.0)
        hcat = jnp.concatenate([h1, h2], axis=1)
        h = jnp.maximum(jnp.dot(hcat, fw_r[...],
                                preferred_element_type=jnp.float32)
                        + fb_r[...], 0.0)
        mu = jnp.dot(h, muw_r[...], preferred_element_type=jnp.float32) \
            + mub_r[...]
        lv = jnp.dot(h, lvw_r[...], preferred_element_type=jnp.float32) \
            + lvb_r[...]
        z_r[...] = mu + eps_r[...] * jnp.exp(0.5 * lv)
        mu_r[...] = mu
        lv_r[...] = lv

    h2 = c1.shape[1]
    fin = 2 * h2
    fus = p['fuse_w'].shape[1]
    zd = p['mu_w'].shape[1]

    def cs(shape):
        return pl.BlockSpec(shape, _const)

    return pl.pallas_call(
        body,
        grid=(GRID,),
        in_specs=[pl.BlockSpec((BLK, h2), _rows), pl.BlockSpec((BLK, h2), _rows),
                  pl.BlockSpec((BLK, 1), _rows), pl.BlockSpec((BLK, zd), _rows),
                  cs((1, h2)), cs((1, h2)),
                  cs((fin, fus)), cs((1, fus)),
                  cs((fus, zd)), cs((1, zd)),
                  cs((fus, zd)), cs((1, zd))],
        out_specs=[pl.BlockSpec((BLK, zd), _rows), pl.BlockSpec((BLK, zd), _rows),
                   pl.BlockSpec((BLK, zd), _rows)],
        out_shape=[jax.ShapeDtypeStruct((NP, zd), jnp.float32)] * 3,
    )(c1, c2, dinv, eps,
      p['gcn1_b2'].reshape(1, h2), p['gcn2_b2'].reshape(1, h2),
      p['fuse_w'], p['fuse_b'].reshape(1, fus),
      p['mu_w'], p['mu_b'].reshape(1, zd),
      p['logvar_w'], p['logvar_b'].reshape(1, zd))


def _tc_decode(z, p):
    def body(z_r, r1w1_r, r1b1_r, r1w2_r, r1b2_r,
             r2w1_r, r2b1_r, r2w2_r, r2b2_r,
             rsw1_r, rsb1_r, rsw2_r, rsb2_r, x1_r, x2_r, xs_r):
        z_b = z_r[...]
        t1 = jnp.maximum(jnp.dot(z_b, r1w1_r[...],
                                 preferred_element_type=jnp.float32)
                         + r1b1_r[...], 0.0)
        x1_r[...] = jnp.dot(t1, r1w2_r[...],
                            preferred_element_type=jnp.float32) + r1b2_r[...]
        t2 = jnp.maximum(jnp.dot(z_b, r2w1_r[...],
                                 preferred_element_type=jnp.float32)
                         + r2b1_r[...], 0.0)
        x2_r[...] = jnp.dot(t2, r2w2_r[...],
                            preferred_element_type=jnp.float32) + r2b2_r[...]
        ts = jnp.maximum(jnp.dot(z_b, rsw1_r[...],
                                 preferred_element_type=jnp.float32)
                         + rsb1_r[...], 0.0)
        xs_r[...] = jnp.dot(ts, rsw2_r[...],
                            preferred_element_type=jnp.float32) + rsb2_r[...]

    fus = p['fuse_w'].shape[1]
    zd = p['mu_w'].shape[1]
    r1 = p['rec1_w2'].shape[1]
    r2 = p['rec2_w2'].shape[1]
    rs = p['recs_w2'].shape[1]

    def cs(shape):
        return pl.BlockSpec(shape, _const)

    return pl.pallas_call(
        body,
        grid=(GRID,),
        in_specs=[pl.BlockSpec((BLK, zd), _rows),
                  cs((zd, fus)), cs((1, fus)), cs((fus, r1)), cs((1, r1)),
                  cs((zd, fus)), cs((1, fus)), cs((fus, r2)), cs((1, r2)),
                  cs((zd, fus)), cs((1, fus)), cs((fus, rs)), cs((1, rs))],
        out_specs=[pl.BlockSpec((BLK, r1), _rows), pl.BlockSpec((BLK, r2), _rows),
                   pl.BlockSpec((BLK, rs), _rows)],
        out_shape=[jax.ShapeDtypeStruct((NP, r1), jnp.float32),
                   jax.ShapeDtypeStruct((NP, r2), jnp.float32),
                   jax.ShapeDtypeStruct((NP, rs), jnp.float32)],
    )(z, p['rec1_w1'], p['rec1_b1'].reshape(1, fus),
      p['rec1_w2'], p['rec1_b2'].reshape(1, r1),
      p['rec2_w1'], p['rec2_b1'].reshape(1, fus),
      p['rec2_w2'], p['rec2_b2'].reshape(1, r2),
      p['recs_w1'], p['recs_b1'].reshape(1, fus),
      p['recs_w2'], p['recs_b2'].reshape(1, rs))


def _tc_edge_dot(zs, zd):
    eb = 2048
    egrid = EP // eb

    def body(zs_r, zd_r, o_r):
        s = jnp.sum(zs_r[...] * zd_r[...], axis=1, keepdims=True)
        o_r[...] = jax.nn.sigmoid(s)

    k = zs.shape[1]
    return pl.pallas_call(
        body,
        grid=(egrid,),
        in_specs=[pl.BlockSpec((eb, k), _rows), pl.BlockSpec((eb, k), _rows)],
        out_specs=pl.BlockSpec((eb, 1), _rows),
        out_shape=jax.ShapeDtypeStruct((EP, 1), jnp.float32),
    )(zs, zd)


def kernel(x_omics1, x_omics2, edge_index, params):
    p = params
    src = edge_index[0].astype(jnp.int32)
    dst = edge_index[1].astype(jnp.int32)
    pad = EP - E
    src2d = jnp.concatenate([src, jnp.zeros((pad,), jnp.int32)])
    src2d = src2d.reshape(EROWSP, LANES)
    dst2d = jnp.concatenate([dst, jnp.full((pad,), N, jnp.int32)])
    dst2d = dst2d.reshape(EROWSP, LANES)
    ones = jnp.ones((LANES, 16), jnp.float32)
    zeros = jnp.zeros((RPTP, 16), jnp.float32)
    e2d = jnp.stack([src2d, dst2d], axis=1)
    s512 = src2d.reshape(EROWSP // 4, 4 * LANES)
    d512 = dst2d.reshape(EROWSP // 4, 4 * LANES)
    eps = jax.random.normal(jax.random.key(42), (N, 32), jnp.float32)

    parts = _sc_degree(dst2d, ones, zeros)
    u1, u2, dinv = _tc_front(parts.reshape(2, NP, 16), x_omics1, x_omics2,
                             p['gcn1_w1'], p['gcn2_w1'])
    a1, a2 = _sc_agg(u1, u2, e2d, u1.shape[1])
    v1, v2 = _tc_mid(a1, a2, dinv, p['gcn1_b1'].reshape(1, -1),
                     p['gcn2_b1'].reshape(1, -1), p['gcn1_w2'], p['gcn2_w2'])
    c1, c2 = _sc_agg2(v1, v2, s512, d512, v1.shape[1])
    z, mu, logvar = _tc_latent(c1, c2, dinv, eps, p)
    zs, zd = _sc_gather_edges(z, s512, d512)
    xh1, xh2, xhs = _tc_decode(z, p)
    adj = _tc_edge_dot(zs, zd).reshape(EP)[:E]
    return (z[:N], mu[:N], logvar[:N], xh1[:N], xh2[:N], xhs[:N], adj)


# speedup vs baseline: 1.0884x; 1.0629x over previous
import functools

import jax
import jax.numpy as jnp
from jax import lax
from jax.experimental import pallas as pl
from jax.experimental.pallas import tpu as pltpu
from jax.experimental.pallas import tpu_sc as plsc

N = 10000
E = 160000
NC = 2
NS = 16
LANES = 128
NP = 10240
EP = 163840
EROWSP = EP // LANES
RPTP = NP // NS
DROWS = EROWSP // (NC * NS)
AROWS = EROWSP // NS
BLK = 512
GRID = NP // BLK


def _mesh():
    return plsc.VectorSubcoreMesh(
        core_axis_name="c", subcore_axis_name="s",
        num_cores=NC, num_subcores=NS)


_SC_PARAMS = pltpu.CompilerParams(use_tc_tiling_on_sc=False,
                                  skip_device_barrier=True)


def _al(x):
    return pl.multiple_of(x, 8)


def _sc_degree(dst2d, ones, zeros):
    @functools.partial(
        pl.kernel,
        out_type=jax.ShapeDtypeStruct((2 * NP, 16), jnp.float32),
        mesh=_mesh(),
        compiler_params=_SC_PARAMS,
        scratch_types=[
            pltpu.VMEM((DROWS, LANES), jnp.int32),
            pltpu.VMEM((LANES, 16), jnp.float32),
            pltpu.VMEM_SHARED((NP, 16), jnp.float32),
        ],
    )
    def k(dst_hbm, ones_hbm, zeros_hbm, out_hbm, idx_v, ones_v, acc):
        cid = lax.axis_index("c")
        sid = lax.axis_index("s")
        w = sid * NC + cid
        pltpu.sync_copy(ones_hbm, ones_v)
        pltpu.sync_copy(dst_hbm.at[pl.ds(_al(w * DROWS), DROWS)], idx_v)
        pltpu.sync_copy(zeros_hbm, acc.at[pl.ds(_al(sid * RPTP), RPTP)])
        plsc.subcore_barrier()

        def body(j, c):
            pltpu.sync_copy(ones_v, acc.at[idx_v.at[j]], add=True)
            return c

        lax.fori_loop(0, DROWS, body, 0)
        plsc.subcore_barrier()
        pltpu.sync_copy(acc.at[pl.ds(_al(sid * RPTP), RPTP)],
                        out_hbm.at[pl.ds(_al(cid * NP + sid * RPTP), RPTP)])

    return k(dst2d, ones, zeros)


def _sc_agg(u1, u2, e2d, wc):
    @functools.partial(
        pl.kernel,
        out_type=(jax.ShapeDtypeStruct((NP, wc), jnp.float32),
                  jax.ShapeDtypeStruct((NP, wc), jnp.float32)),
        mesh=_mesh(),
        compiler_params=_SC_PARAMS,
        scratch_types=[
            pltpu.VMEM((8, 2, LANES), jnp.int32),
            pltpu.VMEM((LANES, wc), jnp.float32),
            pltpu.VMEM((LANES, wc), jnp.float32),
            pltpu.VMEM_SHARED((NP, wc), jnp.float32),
            pltpu.SemaphoreType.DMA,
            pltpu.SemaphoreType.DMA,
        ],
    )
    def k(u1_hbm, u2_hbm, edges, out1, out2, exv, rv0, rv1, acc, sg0, sg1):
        cid = lax.axis_index("c")
        sid = lax.axis_index("s")
        sl = pl.ds(_al(sid * RPTP), RPTP)

        @pl.when(cid == 0)
        def _():
            pltpu.sync_copy(u1_hbm.at[sl], acc.at[sl])

        @pl.when(cid == 1)
        def _():
            pltpu.sync_copy(u2_hbm.at[sl], acc.at[sl])

        plsc.subcore_barrier()
        r0 = sid * AROWS

        def gstart(jj, rv, sg):
            @pl.when(cid == 0)
            def _():
                pltpu.async_copy(u1_hbm.at[exv.at[jj, 0]], rv, sg)

            @pl.when(cid == 1)
            def _():
                pltpu.async_copy(u2_hbm.at[exv.at[jj, 0]], rv, sg)

        def gwait(jj, rv, sg):
            pltpu.make_async_copy(u1_hbm.at[exv.at[jj, 0]], rv, sg).wait()

        def group(g, c):
            pltpu.sync_copy(edges.at[pl.ds(_al(r0 + g * 8), 8)], exv)
            gstart(0, rv0, sg0)

            def pair(k2, c2):
                j0 = 2 * k2
                j1 = j0 + 1
                gstart(j1, rv1, sg1)
                gwait(j0, rv0, sg0)
                pltpu.sync_copy(rv0, acc.at[exv.at[j0, 1]], add=True)

                @pl.when(j0 + 2 < 8)
                def _():
                    gstart(j0 + 2, rv0, sg0)

                gwait(j1, rv1, sg1)
                pltpu.sync_copy(rv1, acc.at[exv.at[j1, 1]], add=True)
                return c2

            lax.fori_loop(0, 4, pair, 0)
            return c

        lax.fori_loop(0, AROWS // 8, group, 0)
        plsc.subcore_barrier()

        @pl.when(cid == 0)
        def _():
            pltpu.sync_copy(acc.at[sl], out1.at[sl])

        @pl.when(cid == 1)
        def _():
            pltpu.sync_copy(acc.at[sl], out2.at[sl])

    return k(u1, u2, e2d)


def _sc_agg2(u1, u2, s2d, d2d, wc):
    W2 = 4 * LANES
    SLABS = AROWS // 4

    @functools.partial(
        pl.kernel,
        out_type=(jax.ShapeDtypeStruct((NP, wc), jnp.float32),
                  jax.ShapeDtypeStruct((NP, wc), jnp.float32)),
        mesh=_mesh(),
        compiler_params=_SC_PARAMS,
        scratch_types=[
            pltpu.VMEM((SLABS, W2), jnp.int32),
            pltpu.VMEM((SLABS, W2), jnp.int32),
            pltpu.VMEM((W2, wc), jnp.float32),
            pltpu.VMEM((W2, wc), jnp.float32),
            pltpu.VMEM_SHARED((NP, wc), jnp.float32),
            pltpu.SemaphoreType.DMA,
            pltpu.SemaphoreType.DMA,
            pltpu.SemaphoreType.DMA,
            pltpu.SemaphoreType.DMA,
        ],
    )
    def k(u1_hbm, u2_hbm, s2d_h, d2d_h, out1, out2, sidx, didx, rv0, rv1,
          acc, sg0, sg1, ss0, ss1):
        cid = lax.axis_index("c")
        sid = lax.axis_index("s")
        sl = pl.ds(_al(sid * RPTP), RPTP)

        @pl.when(cid == 0)
        def _():
            pltpu.sync_copy(u1_hbm.at[sl], acc.at[sl])

        @pl.when(cid == 1)
        def _():
            pltpu.sync_copy(u2_hbm.at[sl], acc.at[sl])

        r0 = _al(sid * SLABS)
        pltpu.sync_copy(s2d_h.at[pl.ds(r0, SLABS)], sidx)
        pltpu.sync_copy(d2d_h.at[pl.ds(r0, SLABS)], didx)
        plsc.subcore_barrier()

        def gstart(j, rv, sg):
            @pl.when(cid == 0)
            def _():
                pltpu.async_copy(u1_hbm.at[sidx.at[j]], rv, sg)

            @pl.when(cid == 1)
            def _():
                pltpu.async_copy(u2_hbm.at[sidx.at[j]], rv, sg)

        def gwait(j, rv, sg):
            pltpu.make_async_copy(u1_hbm.at[sidx.at[j]], rv, sg).wait()

        def sstart(j, rv, ss):
            pltpu.async_copy(rv, acc.at[didx.at[j]], ss, add=True)

        def swait(j, rv, ss):
            pltpu.make_async_copy(rv, acc.at[didx.at[j]], ss).wait()

        gstart(0, rv0, sg0)
        gstart(1, rv1, sg1)

        def pair(k2, c):
            j0 = 2 * k2
            j1 = j0 + 1
            gwait(j0, rv0, sg0)
            sstart(j0, rv0, ss0)
            gwait(j1, rv1, sg1)
            sstart(j1, rv1, ss1)

            @pl.when(j0 + 2 < SLABS)
            def _():
                swait(j0, rv0, ss0)
                gstart(j0 + 2, rv0, sg0)
                swait(j1, rv1, ss1)
                gstart(j1 + 2, rv1, sg1)

            return c

        lax.fori_loop(0, SLABS // 2, pair, 0)
        swait(SLABS - 2, rv0, ss0)
        swait(SLABS - 1, rv1, ss1)
        plsc.subcore_barrier()

        @pl.when(cid == 0)
        def _():
            pltpu.sync_copy(acc.at[sl], out1.at[sl])

        @pl.when(cid == 1)
        def _():
            pltpu.sync_copy(acc.at[sl], out2.at[sl])

    return k(u1, u2, s2d, d2d)


def _sc_gather_edges(z, src2d, dst2d):
    @functools.partial(
        pl.kernel,
        out_type=(jax.ShapeDtypeStruct((EP, 32), jnp.float32),
                  jax.ShapeDtypeStruct((EP, 32), jnp.float32)),
        mesh=_mesh(),
        compiler_params=_SC_PARAMS,
        scratch_types=[
            pltpu.VMEM((DROWS // 4, 4 * LANES), jnp.int32),
            pltpu.VMEM((DROWS // 4, 4 * LANES), jnp.int32),
            pltpu.VMEM((4 * LANES, 32), jnp.float32),
            pltpu.VMEM((4 * LANES, 32), jnp.float32),
            pltpu.VMEM((4 * LANES, 32), jnp.float32),
            pltpu.VMEM((4 * LANES, 32), jnp.float32),
            pltpu.SemaphoreType.DMA,
            pltpu.SemaphoreType.DMA,
            pltpu.SemaphoreType.DMA,
            pltpu.SemaphoreType.DMA,
        ],
    )
    def k(z_hbm, s2d, d2d, zs_out, zd_out, sidx, didx,
          zs0, zd0, zs1, zd1, sa0, sb0, sa1, sb1):
        R = 4
        SLABS = DROWS // R
        cid = lax.axis_index("c")
        sid = lax.axis_index("s")
        w = sid * NC + cid
        r0 = _al(w * SLABS)
        pltpu.sync_copy(s2d.at[pl.ds(r0, SLABS)], sidx)
        pltpu.sync_copy(d2d.at[pl.ds(r0, SLABS)], didx)

        def gstart(j, zsb, zdb, sa, sb):
            pltpu.async_copy(z_hbm.at[sidx.at[j]], zsb, sa)
            pltpu.async_copy(z_hbm.at[didx.at[j]], zdb, sb)

        def drain(j, zsb, zdb, sa, sb):
            pltpu.make_async_copy(z_hbm.at[sidx.at[j]], zsb, sa).wait()
            pltpu.make_async_copy(z_hbm.at[didx.at[j]], zdb, sb).wait()
            esl = pl.ds(_al((r0 + j) * R * LANES), R * LANES)
            pltpu.sync_copy(zsb, zs_out.at[esl])
            pltpu.sync_copy(zdb, zd_out.at[esl])

        gstart(0, zs0, zd0, sa0, sb0)

        def pair(k2, c):
            j0 = 2 * k2
            j1 = j0 + 1
            gstart(j1, zs1, zd1, sa1, sb1)
            drain(j0, zs0, zd0, sa0, sb0)

            @pl.when(j0 + 2 < SLABS)
            def _():
                gstart(j0 + 2, zs0, zd0, sa0, sb0)

            drain(j1, zs1, zd1, sa1, sb1)
            return c

        lax.fori_loop(0, SLABS // 2, pair, 0)

    return k(z, src2d, dst2d)


def _rows(i):
    return (i, 0)


def _const(i):
    return (0, 0)


def _tc_front(parts3, x1, x2, w11, w21):
    def body(p0_r, p1_r, x1_r, w1_r, x2_r, w2_r, u1_r, u2_r, dinv_r):
        deg = p0_r[0, :, 0:1] + p1_r[0, :, 0:1] + 1.0
        dinv = jnp.where(deg > 0, 1.0 / jnp.sqrt(deg), 0.0)
        dinv_r[...] = dinv
        u1_r[...] = jnp.dot(x1_r[...], w1_r[...],
                            preferred_element_type=jnp.float32) * dinv
        u2_r[...] = jnp.dot(x2_r[...], w2_r[...],
                            preferred_element_type=jnp.float32) * dinv

    d = x1.shape[1]
    h = w11.shape[1]
    return pl.pallas_call(
        body,
        grid=(GRID,),
        in_specs=[pl.BlockSpec((1, BLK, 16), lambda i: (0, i, 0)),
                  pl.BlockSpec((1, BLK, 16), lambda i: (1, i, 0)),
                  pl.BlockSpec((BLK, d), _rows), pl.BlockSpec((d, h), _const),
                  pl.BlockSpec((BLK, d), _rows), pl.BlockSpec((d, h), _const)],
        out_specs=[pl.BlockSpec((BLK, h), _rows), pl.BlockSpec((BLK, h), _rows),
                   pl.BlockSpec((BLK, 1), _rows)],
        out_shape=[jax.ShapeDtypeStruct((NP, h), jnp.float32),
                   jax.ShapeDtypeStruct((NP, h), jnp.float32),
                   jax.ShapeDtypeStruct((NP, 1), jnp.float32)],
    )(parts3, parts3, x1, w11, x2, w21)


def _tc_mid(a1, a2, dinv, b11, b21, w12, w22):
    def body(a1_r, a2_r, di_r, b1_r, b2_r, w1_r, w2_r, v1_r, v2_r):
        di = di_r[...]
        g1 = jnp.maximum(a1_r[...] * di + b1_r[...], 0.0)
        g2 = jnp.maximum(a2_r[...] * di + b2_r[...], 0.0)
        v1_r[...] = jnp.dot(g1, w1_r[...],
                            preferred_element_type=jnp.float32) * di
        v2_r[...] = jnp.dot(g2, w2_r[...],
                            preferred_element_type=jnp.float32) * di

    h1 = a1.shape[1]
    h2 = w12.shape[1]
    return pl.pallas_call(
        body,
        grid=(GRID,),
        in_specs=[pl.BlockSpec((BLK, h1), _rows), pl.BlockSpec((BLK, h1), _rows),
                  pl.BlockSpec((BLK, 1), _rows),
                  pl.BlockSpec((1, h1), _const), pl.BlockSpec((1, h1), _const),
                  pl.BlockSpec((h1, h2), _const), pl.BlockSpec((h1, h2), _const)],
        out_specs=[pl.BlockSpec((BLK, h2), _rows), pl.BlockSpec((BLK, h2), _rows)],
        out_shape=[jax.ShapeDtypeStruct((NP, h2), jnp.float32)] * 2,
    )(a1, a2, dinv, b11, b21, w12, w22)


def _tc_latent(c1, c2, dinv, eps, p):
    def body(c1_r, c2_r, di_r, eps_r, b12_r, b22_r, fw_r, fb_r,
             muw_r, mub_r, lvw_r, lvb_r, z_r, mu_r, lv_r):
        di = di_r[...]
        h1 = jnp.maximum(c1_r[...] * di + b12_r[...], 0.0)
        h2 = jnp.maximum(c2_r[...] * di + b22_r[...], 0.0)
        hcat = jnp.concatenate([h1, h2], axis=1)
        h = jnp.maximum(jnp.dot(hcat, fw_r[...],
                                preferred_element_type=jnp.float32)
                        + fb_r[...], 0.0)
        mu = jnp.dot(h, muw_r[...], preferred_element_type=jnp.float32) \
            + mub_r[...]
        lv = jnp.dot(h, lvw_r[...], preferred_element_type=jnp.float32) \
            + lvb_r[...]
        z_r[...] = mu + eps_r[...] * jnp.exp(0.5 * lv)
        mu_r[...] = mu
        lv_r[...] = lv

    h2 = c1.shape[1]
    fin = 2 * h2
    fus = p['fuse_w'].shape[1]
    zd = p['mu_w'].shape[1]

    def cs(shape):
        return pl.BlockSpec(shape, _const)

    return pl.pallas_call(
        body,
        grid=(GRID,),
        in_specs=[pl.BlockSpec((BLK, h2), _rows), pl.BlockSpec((BLK, h2), _rows),
                  pl.BlockSpec((BLK, 1), _rows), pl.BlockSpec((BLK, zd), _rows),
                  cs((1, h2)), cs((1, h2)),
                  cs((fin, fus)), cs((1, fus)),
                  cs((fus, zd)), cs((1, zd)),
                  cs((fus, zd)), cs((1, zd))],
        out_specs=[pl.BlockSpec((BLK, zd), _rows), pl.BlockSpec((BLK, zd), _rows),
                   pl.BlockSpec((BLK, zd), _rows)],
        out_shape=[jax.ShapeDtypeStruct((NP, zd), jnp.float32)] * 3,
    )(c1, c2, dinv, eps,
      p['gcn1_b2'].reshape(1, h2), p['gcn2_b2'].reshape(1, h2),
      p['fuse_w'], p['fuse_b'].reshape(1, fus),
      p['mu_w'], p['mu_b'].reshape(1, zd),
      p['logvar_w'], p['logvar_b'].reshape(1, zd))


def _tc_decode(z, p):
    def body(z_r, r1w1_r, r1b1_r, r1w2_r, r1b2_r,
             r2w1_r, r2b1_r, r2w2_r, r2b2_r,
             rsw1_r, rsb1_r, rsw2_r, rsb2_r, x1_r, x2_r, xs_r):
        z_b = z_r[...]
        t1 = jnp.maximum(jnp.dot(z_b, r1w1_r[...],
                                 preferred_element_type=jnp.float32)
                         + r1b1_r[...], 0.0)
        x1_r[...] = jnp.dot(t1, r1w2_r[...],
                            preferred_element_type=jnp.float32) + r1b2_r[...]
        t2 = jnp.maximum(jnp.dot(z_b, r2w1_r[...],
                                 preferred_element_type=jnp.float32)
                         + r2b1_r[...], 0.0)
        x2_r[...] = jnp.dot(t2, r2w2_r[...],
                            preferred_element_type=jnp.float32) + r2b2_r[...]
        ts = jnp.maximum(jnp.dot(z_b, rsw1_r[...],
                                 preferred_element_type=jnp.float32)
                         + rsb1_r[...], 0.0)
        xs_r[...] = jnp.dot(ts, rsw2_r[...],
                            preferred_element_type=jnp.float32) + rsb2_r[...]

    fus = p['fuse_w'].shape[1]
    zd = p['mu_w'].shape[1]
    r1 = p['rec1_w2'].shape[1]
    r2 = p['rec2_w2'].shape[1]
    rs = p['recs_w2'].shape[1]

    def cs(shape):
        return pl.BlockSpec(shape, _const)

    return pl.pallas_call(
        body,
        grid=(GRID,),
        in_specs=[pl.BlockSpec((BLK, zd), _rows),
                  cs((zd, fus)), cs((1, fus)), cs((fus, r1)), cs((1, r1)),
                  cs((zd, fus)), cs((1, fus)), cs((fus, r2)), cs((1, r2)),
                  cs((zd, fus)), cs((1, fus)), cs((fus, rs)), cs((1, rs))],
        out_specs=[pl.BlockSpec((BLK, r1), _rows), pl.BlockSpec((BLK, r2), _rows),
                   pl.BlockSpec((BLK, rs), _rows)],
        out_shape=[jax.ShapeDtypeStruct((NP, r1), jnp.float32),
                   jax.ShapeDtypeStruct((NP, r2), jnp.float32),
                   jax.ShapeDtypeStruct((NP, rs), jnp.float32)],
    )(z, p['rec1_w1'], p['rec1_b1'].reshape(1, fus),
      p['rec1_w2'], p['rec1_b2'].reshape(1, r1),
      p['rec2_w1'], p['rec2_b1'].reshape(1, fus),
      p['rec2_w2'], p['rec2_b2'].reshape(1, r2),
      p['recs_w1'], p['recs_b1'].reshape(1, fus),
      p['recs_w2'], p['recs_b2'].reshape(1, rs))


def _tc_edge_dot(zs, zd):
    rb = 16
    egrid = EROWSP // rb

    def body(zs_r, zd_r, o_r):
        s = jnp.sum(zs_r[...] * zd_r[...], axis=2)
        o_r[...] = jax.nn.sigmoid(s)

    k = zs.shape[1]
    zs3 = zs.reshape(EROWSP, LANES, k)
    zd3 = zd.reshape(EROWSP, LANES, k)
    return pl.pallas_call(
        body,
        grid=(egrid,),
        in_specs=[pl.BlockSpec((rb, LANES, k), lambda i: (i, 0, 0)),
                  pl.BlockSpec((rb, LANES, k), lambda i: (i, 0, 0))],
        out_specs=pl.BlockSpec((rb, LANES), _rows),
        out_shape=jax.ShapeDtypeStruct((EROWSP, LANES), jnp.float32),
    )(zs3, zd3)


def kernel(x_omics1, x_omics2, edge_index, params):
    p = params
    src = edge_index[0].astype(jnp.int32)
    dst = edge_index[1].astype(jnp.int32)
    pad = EP - E
    src2d = jnp.concatenate([src, jnp.zeros((pad,), jnp.int32)])
    src2d = src2d.reshape(EROWSP, LANES)
    dst2d = jnp.concatenate([dst, jnp.full((pad,), N, jnp.int32)])
    dst2d = dst2d.reshape(EROWSP, LANES)
    ones = jnp.ones((LANES, 16), jnp.float32)
    zeros = jnp.zeros((RPTP, 16), jnp.float32)
    e2d = jnp.stack([src2d, dst2d], axis=1)
    s512 = src2d.reshape(EROWSP // 4, 4 * LANES)
    d512 = dst2d.reshape(EROWSP // 4, 4 * LANES)
    eps = jax.random.normal(jax.random.key(42), (N, 32), jnp.float32)

    parts = _sc_degree(dst2d, ones, zeros)
    u1, u2, dinv = _tc_front(parts.reshape(2, NP, 16), x_omics1, x_omics2,
                             p['gcn1_w1'], p['gcn2_w1'])
    a1, a2 = _sc_agg(u1, u2, e2d, u1.shape[1])
    v1, v2 = _tc_mid(a1, a2, dinv, p['gcn1_b1'].reshape(1, -1),
                     p['gcn2_b1'].reshape(1, -1), p['gcn1_w2'], p['gcn2_w2'])
    c1, c2 = _sc_agg2(v1, v2, s512, d512, v1.shape[1])
    z, mu, logvar = _tc_latent(c1, c2, dinv, eps, p)
    zs, zd = _sc_gather_edges(z, s512, d512)
    xh1, xh2, xhs = _tc_decode(z, p)
    adj = _tc_edge_dot(zs, zd).reshape(EP)[:E]
    return (z[:N], mu[:N], logvar[:N], xh1[:N], xh2[:N], xhs[:N], adj)
